# Initial kernel scaffold; baseline (speedup 1.0000x reference)
import jax
import jax.numpy as jnp
from jax.experimental import pallas as pl


def kernel(local_embeddings, local_labels, weight):
    raise NotImplementedError("write your pallas kernel here")


# speedup vs baseline: 7.2656x; 7.2656x over previous
import functools
import math

import jax
import jax.numpy as jnp
from jax.experimental import pallas as pl
from jax.experimental.pallas import tpu as pltpu

_BATCH = 1024
_EMB = 128
_N = 100000
_S = 64.0
_M2 = 0.5
_EPS = 1e-7

_CB = 2000
_GRID = _N // _CB

_NEG = -1e30
_COS_M = math.cos(_M2)
_SIN_M = math.sin(_M2)
_T_LO = math.cos(math.pi - _M2 - _EPS)
_COS_PI_EPS = math.cos(math.pi - _EPS)
_LOG_CLIP = math.log(1e-30)


def _pfc_kernel(emb_ref, lab_ref, w_ref, loss_ref, ne_scr, m_scr, s_scr, t_scr):
    b = pl.program_id(0)

    @pl.when(b == 0)
    def _init():
        e = emb_ref[...]
        nrm = jnp.sqrt(jnp.sum(e * e, axis=1, keepdims=True))
        ne_scr[...] = e / jnp.maximum(nrm, 1e-12)
        m_scr[...] = jnp.full((_BATCH, 1), _NEG, jnp.float32)
        s_scr[...] = jnp.zeros((_BATCH, 1), jnp.float32)
        t_scr[...] = jnp.zeros((_BATCH, 1), jnp.float32)

    w = w_ref[...]
    wn = jnp.sqrt(jnp.sum(w * w, axis=1, keepdims=True))
    nw = w / jnp.maximum(wn, 1e-12)
    logits = jax.lax.dot_general(
        ne_scr[...], nw,
        (((1,), (1,)), ((), ())),
        preferred_element_type=jnp.float32,
    )
    logits = jnp.clip(logits, -1.0, 1.0)

    cols = b * _CB + jax.lax.broadcasted_iota(jnp.int32, (_BATCH, _CB), 1)
    is_t = cols == lab_ref[...]
    t_scr[...] += jnp.sum(jnp.where(is_t, logits, 0.0), axis=1, keepdims=True)

    x = jnp.where(is_t, _NEG, _S * logits)
    bmax = jnp.max(x, axis=1, keepdims=True)
    m_old = m_scr[...]
    m_new = jnp.maximum(m_old, bmax)
    s_scr[...] = s_scr[...] * jnp.exp(m_old - m_new) + jnp.sum(
        jnp.exp(x - m_new), axis=1, keepdims=True)
    m_scr[...] = m_new

    @pl.when(b == _GRID - 1)
    def _fin():
        t = jnp.clip(t_scr[...], -1.0 + _EPS, 1.0 - _EPS)
        cos_tm = t * _COS_M - jnp.sqrt(jnp.maximum(1.0 - t * t, 0.0)) * _SIN_M
        ft = _S * jnp.where(t < _T_LO, _COS_PI_EPS, cos_tm)
        m = m_scr[...]
        m_mod = jnp.maximum(m, ft)
        s_mod = s_scr[...] * jnp.exp(m - m_mod) + jnp.exp(ft - m_mod)
        logp = ft - m_mod - jnp.log(s_mod)
        logp = jnp.maximum(logp, _LOG_CLIP)
        loss_ref[...] = -jnp.sum(logp, axis=(0, 1), keepdims=True) / _BATCH


@functools.partial(jax.jit, static_argnames=())
def kernel(local_embeddings, local_labels, weight):
    labels = local_labels.astype(jnp.int32).reshape(_BATCH, 1)
    loss = pl.pallas_call(
        _pfc_kernel,
        grid=(_GRID,),
        in_specs=[
            pl.BlockSpec((_BATCH, _EMB), lambda b: (0, 0)),
            pl.BlockSpec((_BATCH, 1), lambda b: (0, 0)),
            pl.BlockSpec((_CB, _EMB), lambda b: (b, 0)),
        ],
        out_specs=pl.BlockSpec((1, 1), lambda b: (0, 0)),
        out_shape=jax.ShapeDtypeStruct((1, 1), jnp.float32),
        scratch_shapes=[
            pltpu.VMEM((_BATCH, _EMB), jnp.float32),
            pltpu.VMEM((_BATCH, 1), jnp.float32),
            pltpu.VMEM((_BATCH, 1), jnp.float32),
            pltpu.VMEM((_BATCH, 1), jnp.float32),
        ],
        compiler_params=pltpu.CompilerParams(
            dimension_semantics=("arbitrary",),
        ),
    )(local_embeddings, labels, weight)
    return loss[0, 0]


# speedup vs baseline: 11.0974x; 1.5274x over previous
import functools
import math

import jax
import jax.numpy as jnp
from jax.experimental import pallas as pl
from jax.experimental.pallas import tpu as pltpu

_BATCH = 1024
_EMB = 128
_N = 100000
_S = 64.0
_M2 = 0.5
_EPS = 1e-7

_CB = 2000
_GRID = _N // _CB

_COS_M = math.cos(_M2)
_SIN_M = math.sin(_M2)
_T_LO = math.cos(math.pi - _M2 - _EPS)
_COS_PI_EPS = math.cos(math.pi - _EPS)
_LOG_CLIP = math.log(1e-30)


def _pfc_kernel(emb_ref, lab_ref, w_ref, loss_ref, ne_scr, s_scr, t_scr):
    b = pl.program_id(0)

    @pl.when(b == 0)
    def _init():
        e = emb_ref[...]
        nrm = jnp.sqrt(jnp.sum(e * e, axis=1, keepdims=True))
        ne_scr[...] = e / jnp.maximum(nrm, 1e-12)
        s_scr[...] = jnp.zeros((_BATCH, 1), jnp.float32)
        t_scr[...] = jnp.zeros((_BATCH, 1), jnp.float32)

    w = w_ref[...]
    wn = jnp.sqrt(jnp.sum(w * w, axis=1, keepdims=True))
    nw = w / jnp.maximum(wn, 1e-12)
    logits = jax.lax.dot_general(
        ne_scr[...], nw,
        (((1,), (1,)), ((), ())),
        preferred_element_type=jnp.float32,
    )
    e = jnp.exp(logits * _S - _S)

    cols = b * _CB + jax.lax.broadcasted_iota(jnp.int32, (_BATCH, _CB), 1)
    is_t = cols == lab_ref[...]
    t_scr[...] += jnp.sum(jnp.where(is_t, logits, 0.0), axis=1, keepdims=True)
    s_scr[...] += jnp.sum(e, axis=1, keepdims=True)

    @pl.when(b == _GRID - 1)
    def _fin():
        t_raw = t_scr[...]
        e_t = jnp.exp(t_raw * _S - _S)
        t = jnp.clip(t_raw, -1.0 + _EPS, 1.0 - _EPS)
        cos_tm = t * _COS_M - jnp.sqrt(jnp.maximum(1.0 - t * t, 0.0)) * _SIN_M
        ft = _S * jnp.where(t < _T_LO, _COS_PI_EPS, cos_tm)
        s_mod = s_scr[...] - e_t + jnp.exp(ft - _S)
        logp = ft - _S - jnp.log(s_mod)
        logp = jnp.maximum(logp, _LOG_CLIP)
        loss_ref[...] = -jnp.sum(logp, axis=(0, 1), keepdims=True) / _BATCH


@functools.partial(jax.jit, static_argnames=())
def kernel(local_embeddings, local_labels, weight):
    labels = local_labels.astype(jnp.int32).reshape(_BATCH, 1)
    loss = pl.pallas_call(
        _pfc_kernel,
        grid=(_GRID,),
        in_specs=[
            pl.BlockSpec((_BATCH, _EMB), lambda b: (0, 0)),
            pl.BlockSpec((_BATCH, 1), lambda b: (0, 0)),
            pl.BlockSpec((_CB, _EMB), lambda b: (b, 0)),
        ],
        out_specs=pl.BlockSpec((1, 1), lambda b: (0, 0)),
        out_shape=jax.ShapeDtypeStruct((1, 1), jnp.float32),
        scratch_shapes=[
            pltpu.VMEM((_BATCH, _EMB), jnp.float32),
            pltpu.VMEM((_BATCH, 1), jnp.float32),
            pltpu.VMEM((_BATCH, 1), jnp.float32),
        ],
        compiler_params=pltpu.CompilerParams(
            dimension_semantics=("arbitrary",),
        ),
    )(local_embeddings, labels, weight)
    return loss[0, 0]


# speedup vs baseline: 11.8731x; 1.0699x over previous
import functools
import math

import jax
import jax.numpy as jnp
from jax import lax
from jax.experimental import pallas as pl
from jax.experimental.pallas import tpu as pltpu
from jax.experimental.pallas import tpu_sc as plsc

_BATCH = 1024
_EMB = 128
_N = 100000
_NPAD = 102400
_S = 64.0
_M2 = 0.5
_EPS = 1e-7

_CB = 2048
_GRID = _NPAD // _CB

_COS_M = math.cos(_M2)
_SIN_M = math.sin(_M2)
_T_LO = math.cos(math.pi - _M2 - _EPS)
_COS_PI_EPS = math.cos(math.pi - _EPS)
_LOG_CLIP = math.log(1e-30)


def _stream_kernel(emb_ref, w_ref, s_ref, ne_scr):
    b = pl.program_id(0)

    @pl.when(b == 0)
    def _init():
        e = emb_ref[...]
        nrm = jnp.sqrt(jnp.sum(e * e, axis=1, keepdims=True))
        ne_scr[...] = (_S * e) / jnp.maximum(nrm, 1e-12)
        s_ref[...] = jnp.zeros((_BATCH, _EMB), jnp.float32)

    w = w_ref[...]
    wn = jnp.sqrt(jnp.sum(w * w, axis=1, keepdims=True))
    nw = w / jnp.maximum(wn, 1e-12)
    l64 = jax.lax.dot_general(
        ne_scr[...], nw,
        (((1,), (1,)), ((), ())),
        preferred_element_type=jnp.float32,
    )
    e = jnp.exp(l64 - _S)
    acc = e[:, 0:_EMB]
    for k in range(1, _CB // _EMB):
        acc = acc + e[:, k * _EMB:(k + 1) * _EMB]
    s_ref[...] += acc


def _epilogue_kernel(emb_ref, g_ref, s_ref, loss_ref):
    e = emb_ref[...]
    nrm = jnp.sqrt(jnp.sum(e * e, axis=1, keepdims=True))
    ne = (_S * e) / jnp.maximum(nrm, 1e-12)
    g = g_ref[...]
    gn = jnp.sqrt(jnp.sum(g * g, axis=1, keepdims=True))
    ng = g / jnp.maximum(gn, 1e-12)
    t64 = jnp.sum(ne * ng, axis=1, keepdims=True)
    s_tot = jnp.sum(s_ref[...], axis=1, keepdims=True)
    e_t = jnp.exp(t64 - _S)
    t = jnp.clip(t64 * (1.0 / _S), -1.0 + _EPS, 1.0 - _EPS)
    cos_tm = t * _COS_M - jnp.sqrt(jnp.maximum(1.0 - t * t, 0.0)) * _SIN_M
    ft = _S * jnp.where(t < _T_LO, _COS_PI_EPS, cos_tm)
    s_mod = s_tot - e_t + jnp.exp(ft - _S)
    logp = ft - _S - jnp.log(s_mod)
    logp = jnp.maximum(logp, _LOG_CLIP)
    loss_ref[...] = -jnp.sum(logp, axis=(0, 1), keepdims=True) / _BATCH


def _make_sc_gather():
    info = plsc.get_sparse_core_info()
    nw_workers = info.num_cores * info.num_subcores
    b_per_w = _BATCH // nw_workers
    mesh = plsc.VectorSubcoreMesh(core_axis_name="c", subcore_axis_name="s")

    @functools.partial(
        pl.kernel, mesh=mesh,
        out_type=jax.ShapeDtypeStruct((_BATCH, _EMB), jnp.float32),
        scratch_types=[
            pltpu.VMEM((b_per_w,), jnp.int32),
            pltpu.VMEM((b_per_w, _EMB), jnp.float32),
            pltpu.SemaphoreType.DMA,
        ],
    )
    def sc_gather(w_hbm, idx_hbm, out_hbm, idx_v, rows_v, sem):
        wid = lax.axis_index("s") * info.num_cores + lax.axis_index("c")
        base = wid * b_per_w
        pltpu.sync_copy(idx_hbm.at[pl.ds(base, b_per_w)], idx_v)
        pltpu.async_copy(w_hbm.at[idx_v], rows_v, sem).wait()
        pltpu.sync_copy(rows_v, out_hbm.at[pl.ds(base, b_per_w)])

    return sc_gather


_sc_gather = _make_sc_gather()


@jax.jit
def kernel(local_embeddings, local_labels, weight):
    labels = local_labels.astype(jnp.int32)
    g = _sc_gather(weight, labels)
    w_pad = jnp.concatenate(
        [weight, jnp.zeros((_NPAD - _N, _EMB), jnp.float32)], axis=0)
    s128 = pl.pallas_call(
        _stream_kernel,
        grid=(_GRID,),
        in_specs=[
            pl.BlockSpec((_BATCH, _EMB), lambda b: (0, 0)),
            pl.BlockSpec((_CB, _EMB), lambda b: (b, 0)),
        ],
        out_specs=pl.BlockSpec((_BATCH, _EMB), lambda b: (0, 0)),
        out_shape=jax.ShapeDtypeStruct((_BATCH, _EMB), jnp.float32),
        scratch_shapes=[
            pltpu.VMEM((_BATCH, _EMB), jnp.float32),
        ],
        compiler_params=pltpu.CompilerParams(
            dimension_semantics=("arbitrary",),
        ),
    )(local_embeddings, w_pad)
    loss = pl.pallas_call(
        _epilogue_kernel,
        out_shape=jax.ShapeDtypeStruct((1, 1), jnp.float32),
    )(local_embeddings, g, s128)
    return loss[0, 0]


# speedup vs baseline: 16.0874x; 1.3549x over previous
import functools
import math

import jax
import jax.numpy as jnp
from jax import lax
from jax.experimental import pallas as pl
from jax.experimental.pallas import tpu as pltpu
from jax.experimental.pallas import tpu_sc as plsc

_BATCH = 1024
_EMB = 128
_N = 100000
_S = 64.0
_M2 = 0.5
_EPS = 1e-7

_CB = 2048
_GRID = (_N + _CB - 1) // _CB

_COS_M = math.cos(_M2)
_SIN_M = math.sin(_M2)
_T_LO = math.cos(math.pi - _M2 - _EPS)
_COS_PI_EPS = math.cos(math.pi - _EPS)
_LOG_CLIP = math.log(1e-30)


def _stream_kernel(emb_ref, w_ref, s_ref, ne_scr):
    b = pl.program_id(0)

    @pl.when(b == 0)
    def _init():
        e = emb_ref[...]
        nrm = jnp.sqrt(jnp.sum(e * e, axis=1, keepdims=True))
        ne_scr[...] = (_S * e) / jnp.maximum(nrm, 1e-12)
        s_ref[...] = jnp.zeros((_BATCH, _EMB), jnp.float32)

    w = w_ref[...]
    wn2 = jnp.sum(w * w, axis=1, keepdims=True)
    nw = w * jax.lax.rsqrt(jnp.maximum(wn2, 1e-24))
    rows = b * _CB + jax.lax.broadcasted_iota(jnp.int32, (_CB, 1), 0)
    nw = jnp.where(rows < _N, nw, 0.0)
    l64 = jax.lax.dot_general(
        ne_scr[...], nw,
        (((1,), (1,)), ((), ())),
        preferred_element_type=jnp.float32,
    )
    e = jnp.exp(l64 - _S)
    acc = e[:, 0:_EMB]
    for k in range(1, _CB // _EMB):
        acc = acc + e[:, k * _EMB:(k + 1) * _EMB]
    s_ref[...] += acc


def _epilogue_kernel(emb_ref, g_ref, s_ref, loss_ref):
    e = emb_ref[...]
    nrm = jnp.sqrt(jnp.sum(e * e, axis=1, keepdims=True))
    ne = (_S * e) / jnp.maximum(nrm, 1e-12)
    g = g_ref[...]
    gn2 = jnp.sum(g * g, axis=1, keepdims=True)
    ng = g * jax.lax.rsqrt(jnp.maximum(gn2, 1e-24))
    t64 = jnp.sum(ne * ng, axis=1, keepdims=True)
    s_tot = jnp.sum(s_ref[...], axis=1, keepdims=True)
    e_t = jnp.exp(t64 - _S)
    t = jnp.clip(t64 * (1.0 / _S), -1.0 + _EPS, 1.0 - _EPS)
    cos_tm = t * _COS_M - jnp.sqrt(jnp.maximum(1.0 - t * t, 0.0)) * _SIN_M
    ft = _S * jnp.where(t < _T_LO, _COS_PI_EPS, cos_tm)
    s_mod = s_tot - e_t + jnp.exp(ft - _S)
    logp = ft - _S - jnp.log(s_mod)
    logp = jnp.maximum(logp, _LOG_CLIP)
    loss_ref[...] = -jnp.sum(logp, axis=(0, 1), keepdims=True) / _BATCH


def _make_sc_gather():
    info = plsc.get_sparse_core_info()
    nw_workers = info.num_cores * info.num_subcores
    b_per_w = _BATCH // nw_workers
    mesh = plsc.VectorSubcoreMesh(core_axis_name="c", subcore_axis_name="s")

    @functools.partial(
        pl.kernel, mesh=mesh,
        out_type=jax.ShapeDtypeStruct((_BATCH, _EMB), jnp.float32),
        scratch_types=[
            pltpu.VMEM((b_per_w,), jnp.int32),
            pltpu.VMEM((b_per_w, _EMB), jnp.float32),
            pltpu.SemaphoreType.DMA,
        ],
    )
    def sc_gather(w_hbm, idx_hbm, out_hbm, idx_v, rows_v, sem):
        wid = lax.axis_index("s") * info.num_cores + lax.axis_index("c")
        base = wid * b_per_w
        pltpu.sync_copy(idx_hbm.at[pl.ds(base, b_per_w)], idx_v)
        pltpu.async_copy(w_hbm.at[idx_v], rows_v, sem).wait()
        pltpu.sync_copy(rows_v, out_hbm.at[pl.ds(base, b_per_w)])

    return sc_gather


_sc_gather = _make_sc_gather()


@jax.jit
def kernel(local_embeddings, local_labels, weight):
    labels = local_labels.astype(jnp.int32)
    g = _sc_gather(weight, labels)
    s128 = pl.pallas_call(
        _stream_kernel,
        grid=(_GRID,),
        in_specs=[
            pl.BlockSpec((_BATCH, _EMB), lambda b: (0, 0)),
            pl.BlockSpec((_CB, _EMB), lambda b: (b, 0)),
        ],
        out_specs=pl.BlockSpec((_BATCH, _EMB), lambda b: (0, 0)),
        out_shape=jax.ShapeDtypeStruct((_BATCH, _EMB), jnp.float32),
        scratch_shapes=[
            pltpu.VMEM((_BATCH, _EMB), jnp.float32),
        ],
        compiler_params=pltpu.CompilerParams(
            dimension_semantics=("arbitrary",),
        ),
    )(local_embeddings, weight)
    loss = pl.pallas_call(
        _epilogue_kernel,
        out_shape=jax.ShapeDtypeStruct((1, 1), jnp.float32),
    )(local_embeddings, g, s128)
    return loss[0, 0]


# speedup vs baseline: 17.2212x; 1.0705x over previous
import functools
import math

import jax
import jax.numpy as jnp
from jax import lax
from jax.experimental import pallas as pl
from jax.experimental.pallas import tpu as pltpu
from jax.experimental.pallas import tpu_sc as plsc

_BATCH = 1024
_EMB = 128
_N = 100000
_S = 64.0
_M2 = 0.5
_EPS = 1e-7

_CB = 2048
_GRID = (_N + _CB - 1) // _CB

_COS_M = math.cos(_M2)
_SIN_M = math.sin(_M2)
_T_LO = math.cos(math.pi - _M2 - _EPS)
_COS_PI_EPS = math.cos(math.pi - _EPS)
_LOG_CLIP = math.log(1e-30)
_LN2 = math.log(2.0)
_C = _S / _LN2
_PAD_ROWS = float(_GRID * _CB - _N)


def _stream_kernel(emb_ref, w_ref, s_ref, ne_scr):
    b = pl.program_id(0)

    @pl.when(b == 0)
    def _init():
        e = emb_ref[...]
        nrm = jnp.sqrt(jnp.sum(e * e, axis=1, keepdims=True))
        ne_scr[...] = (_C * e) / jnp.maximum(nrm, 1e-12)
        s_ref[...] = jnp.zeros((_BATCH, _EMB), jnp.float32)

    w = w_ref[...]
    wn2 = jnp.sum(w * w, axis=1, keepdims=True)
    nw = w * jax.lax.rsqrt(jnp.maximum(wn2, 1e-24))
    rows = b * _CB + jax.lax.broadcasted_iota(jnp.int32, (_CB, 1), 0)
    nw = jnp.where(rows < _N, nw, 0.0)
    l2 = jax.lax.dot_general(
        ne_scr[...], nw,
        (((1,), (1,)), ((), ())),
        preferred_element_type=jnp.float32,
    )
    ev = jnp.exp2(l2)
    acc = ev[:, 0:_EMB]
    for k in range(1, _CB // _EMB):
        acc = acc + ev[:, k * _EMB:(k + 1) * _EMB]
    s_ref[...] += acc


def _epilogue_kernel(emb_ref, g_ref, s_ref, loss_ref):
    e = emb_ref[...]
    nrm = jnp.sqrt(jnp.sum(e * e, axis=1, keepdims=True))
    ne = (_C * e) / jnp.maximum(nrm, 1e-12)
    g = g_ref[...]
    gn2 = jnp.sum(g * g, axis=1, keepdims=True)
    ng = g * jax.lax.rsqrt(jnp.maximum(gn2, 1e-24))
    t2 = jnp.sum(ne * ng, axis=1, keepdims=True)
    s_tot = jnp.sum(s_ref[...], axis=1, keepdims=True) - _PAD_ROWS
    e_t = jnp.exp2(t2)
    t = jnp.clip(t2 * (1.0 / _C), -1.0 + _EPS, 1.0 - _EPS)
    cos_tm = t * _COS_M - jnp.sqrt(jnp.maximum(1.0 - t * t, 0.0)) * _SIN_M
    fc = jnp.where(t < _T_LO, _COS_PI_EPS, cos_tm)
    s_mod = s_tot - e_t + jnp.exp2(_C * fc)
    logp = _S * fc - jnp.log(s_mod)
    logp = jnp.maximum(logp, _LOG_CLIP)
    loss_ref[...] = -jnp.sum(logp, axis=(0, 1), keepdims=True) / _BATCH


def _make_sc_gather():
    info = plsc.get_sparse_core_info()
    nw_workers = info.num_cores * info.num_subcores
    b_per_w = _BATCH // nw_workers
    mesh = plsc.VectorSubcoreMesh(core_axis_name="c", subcore_axis_name="s")

    @functools.partial(
        pl.kernel, mesh=mesh,
        out_type=jax.ShapeDtypeStruct((_BATCH, _EMB), jnp.float32),
        scratch_types=[
            pltpu.VMEM((b_per_w,), jnp.int32),
            pltpu.VMEM((b_per_w, _EMB), jnp.float32),
            pltpu.SemaphoreType.DMA,
        ],
    )
    def sc_gather(w_hbm, idx_hbm, out_hbm, idx_v, rows_v, sem):
        wid = lax.axis_index("s") * info.num_cores + lax.axis_index("c")
        base = wid * b_per_w
        pltpu.sync_copy(idx_hbm.at[pl.ds(base, b_per_w)], idx_v)
        pltpu.async_copy(w_hbm.at[idx_v], rows_v, sem).wait()
        pltpu.sync_copy(rows_v, out_hbm.at[pl.ds(base, b_per_w)])

    return sc_gather


_sc_gather = _make_sc_gather()


@jax.jit
def kernel(local_embeddings, local_labels, weight):
    labels = local_labels.astype(jnp.int32)
    g = _sc_gather(weight, labels)
    s128 = pl.pallas_call(
        _stream_kernel,
        grid=(_GRID,),
        in_specs=[
            pl.BlockSpec((_BATCH, _EMB), lambda b: (0, 0)),
            pl.BlockSpec((_CB, _EMB), lambda b: (b, 0)),
        ],
        out_specs=pl.BlockSpec((_BATCH, _EMB), lambda b: (0, 0)),
        out_shape=jax.ShapeDtypeStruct((_BATCH, _EMB), jnp.float32),
        scratch_shapes=[
            pltpu.VMEM((_BATCH, _EMB), jnp.float32),
        ],
        compiler_params=pltpu.CompilerParams(
            dimension_semantics=("arbitrary",),
        ),
    )(local_embeddings, weight)
    loss = pl.pallas_call(
        _epilogue_kernel,
        out_shape=jax.ShapeDtypeStruct((1, 1), jnp.float32),
    )(local_embeddings, g, s128)
    return loss[0, 0]


# speedup vs baseline: 17.6302x; 1.0238x over previous
import functools
import math

import jax
import jax.numpy as jnp
from jax import lax
from jax.experimental import pallas as pl
from jax.experimental.pallas import tpu as pltpu
from jax.experimental.pallas import tpu_sc as plsc

_BATCH = 1024
_EMB = 128
_N = 100000
_S = 64.0
_M2 = 0.5
_EPS = 1e-7

_CB = 4096
_GRID = (_N + _CB - 1) // _CB

_COS_M = math.cos(_M2)
_SIN_M = math.sin(_M2)
_T_LO = math.cos(math.pi - _M2 - _EPS)
_COS_PI_EPS = math.cos(math.pi - _EPS)
_LOG_CLIP = math.log(1e-30)
_LN2 = math.log(2.0)
_C = _S / _LN2
_PAD_ROWS = float(_GRID * _CB - _N)


def _stream_kernel(emb_ref, w_ref, s_ref, ne_scr):
    b = pl.program_id(0)

    @pl.when(b == 0)
    def _init():
        e = emb_ref[...]
        nrm = jnp.sqrt(jnp.sum(e * e, axis=1, keepdims=True))
        ne_scr[...] = (_C * e) / jnp.maximum(nrm, 1e-12)
        s_ref[...] = jnp.zeros((_BATCH, _EMB), jnp.float32)

    w = w_ref[...]
    wn2 = jnp.sum(w * w, axis=1, keepdims=True)
    nw = w * jax.lax.rsqrt(jnp.maximum(wn2, 1e-24))
    rows = b * _CB + jax.lax.broadcasted_iota(jnp.int32, (_CB, 1), 0)
    nw = jnp.where(rows < _N, nw, 0.0)
    l2 = jax.lax.dot_general(
        ne_scr[...], nw,
        (((1,), (1,)), ((), ())),
        preferred_element_type=jnp.float32,
    )
    ev = jnp.exp2(l2)
    acc = ev[:, 0:_EMB]
    for k in range(1, _CB // _EMB):
        acc = acc + ev[:, k * _EMB:(k + 1) * _EMB]
    s_ref[...] += acc


def _epilogue_kernel(emb_ref, g_ref, s_ref, loss_ref):
    e = emb_ref[...]
    nrm = jnp.sqrt(jnp.sum(e * e, axis=1, keepdims=True))
    ne = (_C * e) / jnp.maximum(nrm, 1e-12)
    g = g_ref[...]
    gn2 = jnp.sum(g * g, axis=1, keepdims=True)
    ng = g * jax.lax.rsqrt(jnp.maximum(gn2, 1e-24))
    t2 = jnp.sum(ne * ng, axis=1, keepdims=True)
    s_tot = jnp.sum(s_ref[...], axis=1, keepdims=True) - _PAD_ROWS
    e_t = jnp.exp2(t2)
    t = jnp.clip(t2 * (1.0 / _C), -1.0 + _EPS, 1.0 - _EPS)
    cos_tm = t * _COS_M - jnp.sqrt(jnp.maximum(1.0 - t * t, 0.0)) * _SIN_M
    fc = jnp.where(t < _T_LO, _COS_PI_EPS, cos_tm)
    s_mod = s_tot - e_t + jnp.exp2(_C * fc)
    logp = _S * fc - jnp.log(s_mod)
    logp = jnp.maximum(logp, _LOG_CLIP)
    loss_ref[...] = -jnp.sum(logp, axis=(0, 1), keepdims=True) / _BATCH


def _make_sc_gather():
    info = plsc.get_sparse_core_info()
    nw_workers = info.num_cores * info.num_subcores
    b_per_w = _BATCH // nw_workers
    mesh = plsc.VectorSubcoreMesh(core_axis_name="c", subcore_axis_name="s")

    @functools.partial(
        pl.kernel, mesh=mesh,
        out_type=jax.ShapeDtypeStruct((_BATCH, _EMB), jnp.float32),
        scratch_types=[
            pltpu.VMEM((b_per_w,), jnp.int32),
            pltpu.VMEM((b_per_w, _EMB), jnp.float32),
            pltpu.SemaphoreType.DMA,
        ],
    )
    def sc_gather(w_hbm, idx_hbm, out_hbm, idx_v, rows_v, sem):
        wid = lax.axis_index("s") * info.num_cores + lax.axis_index("c")
        base = wid * b_per_w
        pltpu.sync_copy(idx_hbm.at[pl.ds(base, b_per_w)], idx_v)
        pltpu.async_copy(w_hbm.at[idx_v], rows_v, sem).wait()
        pltpu.sync_copy(rows_v, out_hbm.at[pl.ds(base, b_per_w)])

    return sc_gather


_sc_gather = _make_sc_gather()


@jax.jit
def kernel(local_embeddings, local_labels, weight):
    labels = local_labels.astype(jnp.int32)
    g = _sc_gather(weight, labels)
    s128 = pl.pallas_call(
        _stream_kernel,
        grid=(_GRID,),
        in_specs=[
            pl.BlockSpec((_BATCH, _EMB), lambda b: (0, 0)),
            pl.BlockSpec((_CB, _EMB), lambda b: (b, 0)),
        ],
        out_specs=pl.BlockSpec((_BATCH, _EMB), lambda b: (0, 0)),
        out_shape=jax.ShapeDtypeStruct((_BATCH, _EMB), jnp.float32),
        scratch_shapes=[
            pltpu.VMEM((_BATCH, _EMB), jnp.float32),
        ],
        compiler_params=pltpu.CompilerParams(
            dimension_semantics=("arbitrary",),
        ),
    )(local_embeddings, weight)
    loss = pl.pallas_call(
        _epilogue_kernel,
        out_shape=jax.ShapeDtypeStruct((1, 1), jnp.float32),
    )(local_embeddings, g, s128)
    return loss[0, 0]
